# baseline (speedup 1.0000x reference)
import functools

import jax
import jax.numpy as jnp
from jax import lax
from jax.experimental import pallas as pl
from jax.experimental.pallas import tpu as pltpu
from jax.experimental.pallas import tpu_sc as plsc

N = 10000
D = 128
H = 128
HID = 256
NUM_LABELS = 7
E = 320000

NC = 2
NS = 16
NW = NC * NS
CK = 128
NBUF = 3
NCHUNK = 81
NCHUNK_T = NCHUNK + NBUF
EPT = CK * NCHUNK
E_REAL = EPT * NW
NDEG = 10240
DEG_PT = NDEG // NS
NROW = 10112
ROWS_PT = NROW // NS

_mesh = plsc.VectorSubcoreMesh(core_axis_name="c", subcore_axis_name="s")


def _zero_vmem_2d(ref, nrows):
    z = jnp.zeros((16,), jnp.float32)

    def body(i, _):
        for j in range(8):
            ref[i, pl.ds(j * 16, 16)] = z
        return 0

    lax.fori_loop(0, nrows, body, 0)


def _zero_vmem_1d(ref, n):
    z = jnp.zeros((16,), jnp.float32)

    def body(i, _):
        ref[pl.ds(i * 16, 16)] = z
        return 0

    lax.fori_loop(0, n // 16, body, 0)


@functools.partial(
    pl.kernel,
    out_type=jax.ShapeDtypeStruct((NC, NDEG), jnp.float32),
    mesh=_mesh,
    scratch_types=[
        pltpu.VMEM_SHARED((NDEG,), jnp.float32),
        pltpu.VMEM((CK,), jnp.int32),
        pltpu.VMEM((CK,), jnp.float32),
        pltpu.VMEM((DEG_PT,), jnp.float32),
    ],
)
def _deg_kernel(rf_hbm, wf_hbm, out, deg_sp, ridx, wbuf, zbuf):
    cid = lax.axis_index("c")
    sid = lax.axis_index("s")
    wid = sid * NC + cid

    _zero_vmem_1d(zbuf, DEG_PT)
    pltpu.sync_copy(zbuf, deg_sp.at[pl.ds(sid * DEG_PT, DEG_PT)])
    plsc.subcore_barrier()

    def chunk(k, _):
        base = (wid * NCHUNK_T + k) * CK
        pltpu.sync_copy(rf_hbm.at[pl.ds(base, CK)], ridx)
        pltpu.sync_copy(wf_hbm.at[pl.ds(base, CK)], wbuf)
        pltpu.sync_copy(wbuf, deg_sp.at[ridx], add=True)
        return 0

    lax.fori_loop(0, NCHUNK, chunk, 0)
    plsc.subcore_barrier()
    pltpu.sync_copy(
        deg_sp.at[pl.ds(sid * DEG_PT, DEG_PT)],
        out.at[cid, pl.ds(sid * DEG_PT, DEG_PT)],
    )


@functools.partial(
    pl.kernel,
    out_type=jax.ShapeDtypeStruct((NC, NROW, H), jnp.float32),
    mesh=_mesh,
    scratch_types=[
        pltpu.VMEM_SHARED((NROW, H), jnp.float32),
        pltpu.VMEM((CK,), jnp.int32),
        pltpu.VMEM((CK,), jnp.int32),
        pltpu.VMEM((CK,), jnp.int32),
        pltpu.VMEM((CK,), jnp.int32),
        pltpu.VMEM((CK + 16,), jnp.float32),
        pltpu.VMEM((NBUF * CK, H), jnp.float32),
        pltpu.SemaphoreType.DMA,
        pltpu.SemaphoreType.DMA,
        pltpu.SemaphoreType.DMA,
    ],
)
def _agg_kernel(hws_hbm, rf_hbm, cf_hbm, wf_hbm, out, acc_sp, cidx0, cidx1,
                cidx2, ridx, wch, rows, sem0, sem1, sem2):
    cid = lax.axis_index("c")
    sid = lax.axis_index("s")
    wid = sid * NC + cid
    cidx = (cidx0, cidx1, cidx2)
    sems = (sem0, sem1, sem2)

    _zero_vmem_2d(rows, 2 * CK)
    base_row = sid * ROWS_PT
    pltpu.sync_copy(rows.at[pl.ds(0, 2 * CK), :],
                    acc_sp.at[pl.ds(base_row, 2 * CK), :])
    pltpu.sync_copy(rows.at[pl.ds(0, 2 * CK), :],
                    acc_sp.at[pl.ds(base_row + 2 * CK, 2 * CK), :])
    pltpu.sync_copy(rows.at[pl.ds(0, ROWS_PT - 4 * CK), :],
                    acc_sp.at[pl.ds(base_row + 4 * CK, ROWS_PT - 4 * CK), :])
    plsc.subcore_barrier()

    tbase = wid * NCHUNK_T * CK

    def rows_at(b):
        return rows.at[pl.ds(b * CK, CK), :]

    def start_gather(k, b):
        pltpu.sync_copy(cf_hbm.at[pl.ds(tbase + k * CK, CK)], cidx[b])
        pltpu.async_copy(hws_hbm.at[cidx[b]], rows_at(b), sems[b])

    def wait_gather(b):
        pltpu.make_async_copy(hws_hbm.at[cidx[b]], rows_at(b), sems[b]).wait()

    for b in range(NBUF):
        start_gather(b, b)

    def chunk3(kk, _):
        for b in range(NBUF):
            k = NBUF * kk + b
            wait_gather(b)
            pltpu.sync_copy(rf_hbm.at[pl.ds(tbase + k * CK, CK)], ridx)
            pltpu.sync_copy(wf_hbm.at[pl.ds(tbase + k * CK, CK)],
                            wch.at[pl.ds(0, CK)])

            def scale(e, _):
                ws = wch[pl.ds(e, 16)][0]
                row = b * CK + e
                for j in range(8):
                    sl = pl.ds(j * 16, 16)
                    rows[row, sl] = rows[row, sl] * ws
                return 0

            lax.fori_loop(0, CK, scale, 0)
            pltpu.sync_copy(rows_at(b), acc_sp.at[ridx], add=True)
            start_gather(k + NBUF, b)
        return 0

    lax.fori_loop(0, NCHUNK // NBUF, chunk3, 0)
    for b in range(NBUF):
        wait_gather(b)
    plsc.subcore_barrier()

    for j in range(4):
        pltpu.sync_copy(acc_sp.at[pl.ds(base_row + j * CK, CK), :],
                        out.at[cid, pl.ds(base_row + j * CK, CK), :])
    rem = ROWS_PT - 4 * CK
    pltpu.sync_copy(acc_sp.at[pl.ds(base_row + 4 * CK, rem), :],
                    out.at[cid, pl.ds(base_row + 4 * CK, rem), :])


RB = 400
GRID = N // RB


def _mm1_body(x_ref, w_ref, d0_ref, d1_ref, hws_ref, dinv_ref):
    dinv = lax.rsqrt(d0_ref[...] + d1_ref[...] + 1.0)
    hw = jnp.dot(x_ref[...], w_ref[...], preferred_element_type=jnp.float32)
    hws_ref[...] = dinv * hw
    dinv_ref[...] = dinv


def _mm1(x, W0, d0, d1):
    return pl.pallas_call(
        _mm1_body,
        grid=(GRID,),
        in_specs=[
            pl.BlockSpec((RB, D), lambda i: (i, 0)),
            pl.BlockSpec((D, H), lambda i: (0, 0)),
            pl.BlockSpec((RB, 1), lambda i: (i, 0)),
            pl.BlockSpec((RB, 1), lambda i: (i, 0)),
        ],
        out_specs=[
            pl.BlockSpec((RB, H), lambda i: (i, 0)),
            pl.BlockSpec((RB, 1), lambda i: (i, 0)),
        ],
        out_shape=[
            jax.ShapeDtypeStruct((N, H), jnp.float32),
            jax.ShapeDtypeStruct((N, 1), jnp.float32),
        ],
    )(x, W0, d0, d1)


def _combine_mm_body(p0_ref, p1_ref, hws_ref, dinv_ref, b_ref, w_ref, out_ref):
    dinv = dinv_ref[...]
    h = jax.nn.relu(dinv * (p0_ref[0] + p1_ref[0] + hws_ref[...])
                    + b_ref[...])
    out_ref[...] = dinv * jnp.dot(h, w_ref[...],
                                  preferred_element_type=jnp.float32)


def _combine_mm(p, hws, dinv, b, W):
    return pl.pallas_call(
        _combine_mm_body,
        grid=(GRID,),
        in_specs=[
            pl.BlockSpec((1, RB, H), lambda i: (0, i, 0)),
            pl.BlockSpec((1, RB, H), lambda i: (1, i, 0)),
            pl.BlockSpec((RB, H), lambda i: (i, 0)),
            pl.BlockSpec((RB, 1), lambda i: (i, 0)),
            pl.BlockSpec((1, H), lambda i: (0, 0)),
            pl.BlockSpec((H, H), lambda i: (0, 0)),
        ],
        out_specs=pl.BlockSpec((RB, H), lambda i: (i, 0)),
        out_shape=jax.ShapeDtypeStruct((N, H), jnp.float32),
    )(p, p, hws, dinv, b, W)


def _final_body(p0_ref, p1_ref, hws_ref, dinv_ref, b_ref, wd1_ref, bd1_ref,
                wd2_ref, bd2_ref, out_ref):
    dinv = dinv_ref[...]
    h = jax.nn.relu(dinv * (p0_ref[0] + p1_ref[0] + hws_ref[...])
                    + b_ref[...])
    t = jax.nn.relu(jnp.dot(h, wd1_ref[...],
                            preferred_element_type=jnp.float32) + bd1_ref[...])
    out_ref[...] = jnp.dot(t, wd2_ref[...],
                           preferred_element_type=jnp.float32) + bd2_ref[...]


def _final(p, hws, dinv, b, Wd1, bd1, Wd2p, bd2p):
    return pl.pallas_call(
        _final_body,
        grid=(GRID,),
        in_specs=[
            pl.BlockSpec((1, RB, H), lambda i: (0, i, 0)),
            pl.BlockSpec((1, RB, H), lambda i: (1, i, 0)),
            pl.BlockSpec((RB, H), lambda i: (i, 0)),
            pl.BlockSpec((RB, 1), lambda i: (i, 0)),
            pl.BlockSpec((1, H), lambda i: (0, 0)),
            pl.BlockSpec((H, HID), lambda i: (0, 0)),
            pl.BlockSpec((1, HID), lambda i: (0, 0)),
            pl.BlockSpec((HID, H), lambda i: (0, 0)),
            pl.BlockSpec((1, H), lambda i: (0, 0)),
        ],
        out_specs=pl.BlockSpec((RB, H), lambda i: (i, 0)),
        out_shape=jax.ShapeDtypeStruct((N, H), jnp.float32),
    )(p, p, hws, dinv, b, Wd1, bd1, Wd2p, bd2p)


@jax.jit
def kernel(x, edge_index, edge_weight, W0, b0, W1, b1, W2, b2, Wd1, bd1,
           Wd2, bd2):
    r = edge_index[0].astype(jnp.int32)
    c = edge_index[1].astype(jnp.int32)
    w = edge_weight.astype(jnp.float32)
    pad = E_REAL - E
    rf = jnp.pad(jnp.pad(r, (0, pad)).reshape(NW, NCHUNK, CK),
                 ((0, 0), (0, NBUF), (0, 0))).reshape(-1)
    cf = jnp.pad(jnp.pad(c, (0, pad)).reshape(NW, NCHUNK, CK),
                 ((0, 0), (0, NBUF), (0, 0))).reshape(-1)
    wf = jnp.pad(jnp.pad(w, (0, pad)).reshape(NW, NCHUNK, CK),
                 ((0, 0), (0, NBUF), (0, 0))).reshape(-1)

    degp = _deg_kernel(rf, wf)
    d0 = degp[0, :N, None]
    d1 = degp[1, :N, None]

    hws, dinv = _mm1(x, W0, d0, d1)

    p = _agg_kernel(hws, rf, cf, wf)
    hws = _combine_mm(p, hws, dinv, b0.reshape(1, H), W1)

    p = _agg_kernel(hws, rf, cf, wf)
    hws = _combine_mm(p, hws, dinv, b1.reshape(1, H), W2)

    p = _agg_kernel(hws, rf, cf, wf)
    Wd2p = jnp.pad(Wd2, ((0, 0), (0, H - NUM_LABELS)))
    bd2p = jnp.pad(bd2, (0, H - NUM_LABELS)).reshape(1, H)
    out = _final(p, hws, dinv, b2.reshape(1, H), Wd1,
                 bd1.reshape(1, HID), Wd2p, bd2p)
    return out[:, :NUM_LABELS]


# speedup vs baseline: 1.2209x; 1.2209x over previous
import functools

import jax
import jax.numpy as jnp
from jax import lax
from jax.experimental import pallas as pl
from jax.experimental.pallas import tpu as pltpu
from jax.experimental.pallas import tpu_sc as plsc

N = 10000
D = 128
H = 128
HID = 256
NUM_LABELS = 7
E = 320000

NC = 2
NS = 16
NW = NC * NS
CK = 128
NBUF = 3
NCHUNK = 81
NCHUNK_T = NCHUNK + NBUF
EPT = CK * NCHUNK
E_REAL = EPT * NW
NDEG = 10240
DEG_PT = NDEG // NS
NROW = 10112
ROWS_PT = NROW // NS

_mesh = plsc.VectorSubcoreMesh(core_axis_name="c", subcore_axis_name="s")


def _zero_vmem_2d(ref, nrows):
    z = jnp.zeros((16,), jnp.float32)

    def body(i, _):
        for j in range(8):
            ref[i, pl.ds(j * 16, 16)] = z
        return 0

    lax.fori_loop(0, nrows, body, 0)


def _zero_vmem_1d(ref, n):
    z = jnp.zeros((16,), jnp.float32)

    def body(i, _):
        ref[pl.ds(i * 16, 16)] = z
        return 0

    lax.fori_loop(0, n // 16, body, 0)


@functools.partial(
    pl.kernel,
    out_type=jax.ShapeDtypeStruct((NC, NDEG), jnp.float32),
    mesh=_mesh,
    scratch_types=[
        pltpu.VMEM_SHARED((NDEG,), jnp.float32),
        pltpu.VMEM((CK,), jnp.int32),
        pltpu.VMEM((CK,), jnp.float32),
        pltpu.VMEM((DEG_PT,), jnp.float32),
    ],
)
def _deg_kernel(rf_hbm, wf_hbm, out, deg_sp, ridx, wbuf, zbuf):
    cid = lax.axis_index("c")
    sid = lax.axis_index("s")
    wid = sid * NC + cid

    _zero_vmem_1d(zbuf, DEG_PT)
    pltpu.sync_copy(zbuf, deg_sp.at[pl.ds(sid * DEG_PT, DEG_PT)])
    plsc.subcore_barrier()

    def chunk(k, _):
        base = (wid * NCHUNK_T + k) * CK
        pltpu.sync_copy(rf_hbm.at[pl.ds(base, CK)], ridx)
        pltpu.sync_copy(wf_hbm.at[pl.ds(base, CK)], wbuf)
        pltpu.sync_copy(wbuf, deg_sp.at[ridx], add=True)
        return 0

    lax.fori_loop(0, NCHUNK, chunk, 0)
    plsc.subcore_barrier()
    pltpu.sync_copy(
        deg_sp.at[pl.ds(sid * DEG_PT, DEG_PT)],
        out.at[cid, pl.ds(sid * DEG_PT, DEG_PT)],
    )


@functools.partial(
    pl.kernel,
    out_type=jax.ShapeDtypeStruct((NC, NROW, H), jnp.float32),
    mesh=_mesh,
    scratch_types=[
        pltpu.VMEM_SHARED((NROW, H), jnp.float32),
        pltpu.VMEM((CK,), jnp.int32),
        pltpu.VMEM((CK,), jnp.int32),
        pltpu.VMEM((CK,), jnp.int32),
        pltpu.VMEM((CK,), jnp.int32),
        pltpu.VMEM((CK + 16,), jnp.float32),
        pltpu.VMEM((NBUF * CK, H), jnp.float32),
        pltpu.SemaphoreType.DMA,
        pltpu.SemaphoreType.DMA,
        pltpu.SemaphoreType.DMA,
    ],
)
def _agg_kernel(hws_hbm, rf_hbm, cf_hbm, wf_hbm, out, acc_sp, cidx0, cidx1,
                cidx2, ridx, wch, rows, sem0, sem1, sem2):
    cid = lax.axis_index("c")
    sid = lax.axis_index("s")
    wid = sid * NC + cid
    cidx = (cidx0, cidx1, cidx2)
    sems = (sem0, sem1, sem2)

    _zero_vmem_2d(rows, 2 * CK)
    base_row = sid * ROWS_PT
    pltpu.sync_copy(rows.at[pl.ds(0, 2 * CK), :],
                    acc_sp.at[pl.ds(base_row, 2 * CK), :])
    pltpu.sync_copy(rows.at[pl.ds(0, 2 * CK), :],
                    acc_sp.at[pl.ds(base_row + 2 * CK, 2 * CK), :])
    pltpu.sync_copy(rows.at[pl.ds(0, ROWS_PT - 4 * CK), :],
                    acc_sp.at[pl.ds(base_row + 4 * CK, ROWS_PT - 4 * CK), :])
    plsc.subcore_barrier()

    tbase = wid * NCHUNK_T * CK

    def rows_at(b):
        return rows.at[pl.ds(b * CK, CK), :]

    def chunk(k, _):
        base = tbase + k * CK
        pltpu.sync_copy(cf_hbm.at[pl.ds(base, CK)], cidx0)
        pltpu.sync_copy(hws_hbm.at[cidx0], rows_at(0))
        pltpu.sync_copy(rf_hbm.at[pl.ds(base, CK)], ridx)
        pltpu.sync_copy(wf_hbm.at[pl.ds(base, CK)], wch.at[pl.ds(0, CK)])

        def scale(e, _):
            ws = wch[pl.ds(e, 16)][0]
            for j in range(8):
                sl = pl.ds(j * 16, 16)
                rows[e, sl] = rows[e, sl] * ws
            return 0

        lax.fori_loop(0, CK, scale, 0)
        pltpu.sync_copy(rows_at(0), acc_sp.at[ridx], add=True)
        return 0

    lax.fori_loop(0, NCHUNK, chunk, 0)
    plsc.subcore_barrier()

    for j in range(4):
        pltpu.sync_copy(acc_sp.at[pl.ds(base_row + j * CK, CK), :],
                        out.at[cid, pl.ds(base_row + j * CK, CK), :])
    rem = ROWS_PT - 4 * CK
    pltpu.sync_copy(acc_sp.at[pl.ds(base_row + 4 * CK, rem), :],
                    out.at[cid, pl.ds(base_row + 4 * CK, rem), :])


RB = 400
GRID = N // RB


def _mm1_body(x_ref, w_ref, d0_ref, d1_ref, hws_ref, dinv_ref):
    dinv = lax.rsqrt(d0_ref[...] + d1_ref[...] + 1.0)
    hw = jnp.dot(x_ref[...], w_ref[...], preferred_element_type=jnp.float32)
    hws_ref[...] = dinv * hw
    dinv_ref[...] = dinv


def _mm1(x, W0, d0, d1):
    return pl.pallas_call(
        _mm1_body,
        grid=(GRID,),
        in_specs=[
            pl.BlockSpec((RB, D), lambda i: (i, 0)),
            pl.BlockSpec((D, H), lambda i: (0, 0)),
            pl.BlockSpec((RB, 1), lambda i: (i, 0)),
            pl.BlockSpec((RB, 1), lambda i: (i, 0)),
        ],
        out_specs=[
            pl.BlockSpec((RB, H), lambda i: (i, 0)),
            pl.BlockSpec((RB, 1), lambda i: (i, 0)),
        ],
        out_shape=[
            jax.ShapeDtypeStruct((N, H), jnp.float32),
            jax.ShapeDtypeStruct((N, 1), jnp.float32),
        ],
    )(x, W0, d0, d1)


def _combine_mm_body(p0_ref, p1_ref, hws_ref, dinv_ref, b_ref, w_ref, out_ref):
    dinv = dinv_ref[...]
    h = jax.nn.relu(dinv * (p0_ref[0] + p1_ref[0] + hws_ref[...])
                    + b_ref[...])
    out_ref[...] = dinv * jnp.dot(h, w_ref[...],
                                  preferred_element_type=jnp.float32)


def _combine_mm(p, hws, dinv, b, W):
    return pl.pallas_call(
        _combine_mm_body,
        grid=(GRID,),
        in_specs=[
            pl.BlockSpec((1, RB, H), lambda i: (0, i, 0)),
            pl.BlockSpec((1, RB, H), lambda i: (1, i, 0)),
            pl.BlockSpec((RB, H), lambda i: (i, 0)),
            pl.BlockSpec((RB, 1), lambda i: (i, 0)),
            pl.BlockSpec((1, H), lambda i: (0, 0)),
            pl.BlockSpec((H, H), lambda i: (0, 0)),
        ],
        out_specs=pl.BlockSpec((RB, H), lambda i: (i, 0)),
        out_shape=jax.ShapeDtypeStruct((N, H), jnp.float32),
    )(p, p, hws, dinv, b, W)


def _final_body(p0_ref, p1_ref, hws_ref, dinv_ref, b_ref, wd1_ref, bd1_ref,
                wd2_ref, bd2_ref, out_ref):
    dinv = dinv_ref[...]
    h = jax.nn.relu(dinv * (p0_ref[0] + p1_ref[0] + hws_ref[...])
                    + b_ref[...])
    t = jax.nn.relu(jnp.dot(h, wd1_ref[...],
                            preferred_element_type=jnp.float32) + bd1_ref[...])
    out_ref[...] = jnp.dot(t, wd2_ref[...],
                           preferred_element_type=jnp.float32) + bd2_ref[...]


def _final(p, hws, dinv, b, Wd1, bd1, Wd2p, bd2p):
    return pl.pallas_call(
        _final_body,
        grid=(GRID,),
        in_specs=[
            pl.BlockSpec((1, RB, H), lambda i: (0, i, 0)),
            pl.BlockSpec((1, RB, H), lambda i: (1, i, 0)),
            pl.BlockSpec((RB, H), lambda i: (i, 0)),
            pl.BlockSpec((RB, 1), lambda i: (i, 0)),
            pl.BlockSpec((1, H), lambda i: (0, 0)),
            pl.BlockSpec((H, HID), lambda i: (0, 0)),
            pl.BlockSpec((1, HID), lambda i: (0, 0)),
            pl.BlockSpec((HID, H), lambda i: (0, 0)),
            pl.BlockSpec((1, H), lambda i: (0, 0)),
        ],
        out_specs=pl.BlockSpec((RB, H), lambda i: (i, 0)),
        out_shape=jax.ShapeDtypeStruct((N, H), jnp.float32),
    )(p, p, hws, dinv, b, Wd1, bd1, Wd2p, bd2p)


@jax.jit
def kernel(x, edge_index, edge_weight, W0, b0, W1, b1, W2, b2, Wd1, bd1,
           Wd2, bd2):
    r = edge_index[0].astype(jnp.int32)
    c = edge_index[1].astype(jnp.int32)
    w = edge_weight.astype(jnp.float32)
    pad = E_REAL - E
    rf = jnp.pad(jnp.pad(r, (0, pad)).reshape(NW, NCHUNK, CK),
                 ((0, 0), (0, NBUF), (0, 0))).reshape(-1)
    cf = jnp.pad(jnp.pad(c, (0, pad)).reshape(NW, NCHUNK, CK),
                 ((0, 0), (0, NBUF), (0, 0))).reshape(-1)
    wf = jnp.pad(jnp.pad(w, (0, pad)).reshape(NW, NCHUNK, CK),
                 ((0, 0), (0, NBUF), (0, 0))).reshape(-1)

    degp = _deg_kernel(rf, wf)
    d0 = degp[0, :N, None]
    d1 = degp[1, :N, None]

    hws, dinv = _mm1(x, W0, d0, d1)

    p = _agg_kernel(hws, rf, cf, wf)
    hws = _combine_mm(p, hws, dinv, b0.reshape(1, H), W1)

    p = _agg_kernel(hws, rf, cf, wf)
    hws = _combine_mm(p, hws, dinv, b1.reshape(1, H), W2)

    p = _agg_kernel(hws, rf, cf, wf)
    Wd2p = jnp.pad(Wd2, ((0, 0), (0, H - NUM_LABELS)))
    bd2p = jnp.pad(bd2, (0, H - NUM_LABELS)).reshape(1, H)
    out = _final(p, hws, dinv, b2.reshape(1, H), Wd1,
                 bd1.reshape(1, HID), Wd2p, bd2p)
    return out[:, :NUM_LABELS]


# speedup vs baseline: 1.3031x; 1.0673x over previous
import functools

import jax
import jax.numpy as jnp
from jax import lax
from jax.experimental import pallas as pl
from jax.experimental.pallas import tpu as pltpu
from jax.experimental.pallas import tpu_sc as plsc

N = 10000
D = 128
H = 128
HID = 256
NUM_LABELS = 7
E = 320000

NC = 2
NS = 16
NW = NC * NS
CK = 128
NBUF = 3
NCHUNK = 81
NCHUNK_T = NCHUNK + NBUF
EPT = CK * NCHUNK
E_REAL = EPT * NW
NDEG = 10240
DEG_PT = NDEG // NS
NROW = 10112
ROWS_PT = NROW // NS

_mesh = plsc.VectorSubcoreMesh(core_axis_name="c", subcore_axis_name="s")


def _zero_vmem_2d(ref, nrows):
    z = jnp.zeros((16,), jnp.float32)

    def body(i, _):
        for j in range(8):
            ref[i, pl.ds(j * 16, 16)] = z
        return 0

    lax.fori_loop(0, nrows, body, 0)


def _zero_vmem_1d(ref, n):
    z = jnp.zeros((16,), jnp.float32)

    def body(i, _):
        ref[pl.ds(i * 16, 16)] = z
        return 0

    lax.fori_loop(0, n // 16, body, 0)


@functools.partial(
    pl.kernel,
    out_type=jax.ShapeDtypeStruct((NC, NDEG), jnp.float32),
    mesh=_mesh,
    scratch_types=[
        pltpu.VMEM_SHARED((NDEG,), jnp.float32),
        pltpu.VMEM((CK,), jnp.int32),
        pltpu.VMEM((CK,), jnp.float32),
        pltpu.VMEM((DEG_PT,), jnp.float32),
    ],
)
def _deg_kernel(rf_hbm, wf_hbm, out, deg_sp, ridx, wbuf, zbuf):
    cid = lax.axis_index("c")
    sid = lax.axis_index("s")
    wid = sid * NC + cid

    _zero_vmem_1d(zbuf, DEG_PT)
    pltpu.sync_copy(zbuf, deg_sp.at[pl.ds(sid * DEG_PT, DEG_PT)])
    plsc.subcore_barrier()

    def chunk(k, _):
        base = (wid * NCHUNK_T + k) * CK
        pltpu.sync_copy(rf_hbm.at[pl.ds(base, CK)], ridx)
        pltpu.sync_copy(wf_hbm.at[pl.ds(base, CK)], wbuf)
        pltpu.sync_copy(wbuf, deg_sp.at[ridx], add=True)
        return 0

    lax.fori_loop(0, NCHUNK, chunk, 0)
    plsc.subcore_barrier()
    pltpu.sync_copy(
        deg_sp.at[pl.ds(sid * DEG_PT, DEG_PT)],
        out.at[cid, pl.ds(sid * DEG_PT, DEG_PT)],
    )


@functools.partial(
    pl.kernel,
    out_type=jax.ShapeDtypeStruct((NC, NROW, H), jnp.float32),
    mesh=_mesh,
    scratch_types=[
        pltpu.VMEM_SHARED((NROW, H), jnp.float32),
        pltpu.VMEM((CK,), jnp.int32),
        pltpu.VMEM((CK,), jnp.int32),
        pltpu.VMEM((CK,), jnp.int32),
        pltpu.VMEM((CK,), jnp.int32),
        pltpu.VMEM((CK + 16,), jnp.float32),
        pltpu.VMEM((NBUF * CK, H), jnp.float32),
        pltpu.SemaphoreType.DMA,
        pltpu.SemaphoreType.DMA,
        pltpu.SemaphoreType.DMA,
    ],
)
def _agg_kernel(hws_hbm, rf_hbm, cf_hbm, wf_hbm, out, acc_sp, cidx0, cidx1,
                cidx2, ridx, wch, rows, sem0, sem1, sem2):
    cid = lax.axis_index("c")
    sid = lax.axis_index("s")
    wid = sid * NC + cid
    cidx = (cidx0, cidx1, cidx2)
    sems = (sem0, sem1, sem2)

    _zero_vmem_2d(rows, 2 * CK)
    base_row = sid * ROWS_PT
    pltpu.sync_copy(rows.at[pl.ds(0, 2 * CK), :],
                    acc_sp.at[pl.ds(base_row, 2 * CK), :])
    pltpu.sync_copy(rows.at[pl.ds(0, 2 * CK), :],
                    acc_sp.at[pl.ds(base_row + 2 * CK, 2 * CK), :])
    pltpu.sync_copy(rows.at[pl.ds(0, ROWS_PT - 4 * CK), :],
                    acc_sp.at[pl.ds(base_row + 4 * CK, ROWS_PT - 4 * CK), :])
    plsc.subcore_barrier()

    tbase = wid * NCHUNK_T * CK

    def rows_at(b):
        return rows.at[pl.ds(b * CK, CK), :]

    def chunk(k, _):
        base = tbase + k * CK
        pltpu.sync_copy(cf_hbm.at[pl.ds(base, CK)], cidx0)
        pltpu.async_copy(hws_hbm.at[cidx0], rows_at(0), sem0)
        pltpu.sync_copy(rf_hbm.at[pl.ds(base, CK)], ridx)
        pltpu.sync_copy(wf_hbm.at[pl.ds(base, CK)], wch.at[pl.ds(0, CK)])
        pltpu.make_async_copy(hws_hbm.at[cidx0], rows_at(0), sem0).wait()

        def scale(e, _):
            ws = wch[pl.ds(e, 16)][0]
            for j in range(8):
                sl = pl.ds(j * 16, 16)
                rows[e, sl] = rows[e, sl] * ws
            return 0

        lax.fori_loop(0, CK, scale, 0)
        pltpu.sync_copy(rows_at(0), acc_sp.at[ridx], add=True)
        return 0

    lax.fori_loop(0, NCHUNK, chunk, 0)
    plsc.subcore_barrier()

    for j in range(4):
        pltpu.sync_copy(acc_sp.at[pl.ds(base_row + j * CK, CK), :],
                        out.at[cid, pl.ds(base_row + j * CK, CK), :])
    rem = ROWS_PT - 4 * CK
    pltpu.sync_copy(acc_sp.at[pl.ds(base_row + 4 * CK, rem), :],
                    out.at[cid, pl.ds(base_row + 4 * CK, rem), :])


RB = 400
GRID = N // RB


def _mm1_body(x_ref, w_ref, d0_ref, d1_ref, hws_ref, dinv_ref):
    dinv = lax.rsqrt(d0_ref[...] + d1_ref[...] + 1.0)
    hw = jnp.dot(x_ref[...], w_ref[...], preferred_element_type=jnp.float32)
    hws_ref[...] = dinv * hw
    dinv_ref[...] = dinv


def _mm1(x, W0, d0, d1):
    return pl.pallas_call(
        _mm1_body,
        grid=(GRID,),
        in_specs=[
            pl.BlockSpec((RB, D), lambda i: (i, 0)),
            pl.BlockSpec((D, H), lambda i: (0, 0)),
            pl.BlockSpec((RB, 1), lambda i: (i, 0)),
            pl.BlockSpec((RB, 1), lambda i: (i, 0)),
        ],
        out_specs=[
            pl.BlockSpec((RB, H), lambda i: (i, 0)),
            pl.BlockSpec((RB, 1), lambda i: (i, 0)),
        ],
        out_shape=[
            jax.ShapeDtypeStruct((N, H), jnp.float32),
            jax.ShapeDtypeStruct((N, 1), jnp.float32),
        ],
    )(x, W0, d0, d1)


def _combine_mm_body(p0_ref, p1_ref, hws_ref, dinv_ref, b_ref, w_ref, out_ref):
    dinv = dinv_ref[...]
    h = jax.nn.relu(dinv * (p0_ref[0] + p1_ref[0] + hws_ref[...])
                    + b_ref[...])
    out_ref[...] = dinv * jnp.dot(h, w_ref[...],
                                  preferred_element_type=jnp.float32)


def _combine_mm(p, hws, dinv, b, W):
    return pl.pallas_call(
        _combine_mm_body,
        grid=(GRID,),
        in_specs=[
            pl.BlockSpec((1, RB, H), lambda i: (0, i, 0)),
            pl.BlockSpec((1, RB, H), lambda i: (1, i, 0)),
            pl.BlockSpec((RB, H), lambda i: (i, 0)),
            pl.BlockSpec((RB, 1), lambda i: (i, 0)),
            pl.BlockSpec((1, H), lambda i: (0, 0)),
            pl.BlockSpec((H, H), lambda i: (0, 0)),
        ],
        out_specs=pl.BlockSpec((RB, H), lambda i: (i, 0)),
        out_shape=jax.ShapeDtypeStruct((N, H), jnp.float32),
    )(p, p, hws, dinv, b, W)


def _final_body(p0_ref, p1_ref, hws_ref, dinv_ref, b_ref, wd1_ref, bd1_ref,
                wd2_ref, bd2_ref, out_ref):
    dinv = dinv_ref[...]
    h = jax.nn.relu(dinv * (p0_ref[0] + p1_ref[0] + hws_ref[...])
                    + b_ref[...])
    t = jax.nn.relu(jnp.dot(h, wd1_ref[...],
                            preferred_element_type=jnp.float32) + bd1_ref[...])
    out_ref[...] = jnp.dot(t, wd2_ref[...],
                           preferred_element_type=jnp.float32) + bd2_ref[...]


def _final(p, hws, dinv, b, Wd1, bd1, Wd2p, bd2p):
    return pl.pallas_call(
        _final_body,
        grid=(GRID,),
        in_specs=[
            pl.BlockSpec((1, RB, H), lambda i: (0, i, 0)),
            pl.BlockSpec((1, RB, H), lambda i: (1, i, 0)),
            pl.BlockSpec((RB, H), lambda i: (i, 0)),
            pl.BlockSpec((RB, 1), lambda i: (i, 0)),
            pl.BlockSpec((1, H), lambda i: (0, 0)),
            pl.BlockSpec((H, HID), lambda i: (0, 0)),
            pl.BlockSpec((1, HID), lambda i: (0, 0)),
            pl.BlockSpec((HID, H), lambda i: (0, 0)),
            pl.BlockSpec((1, H), lambda i: (0, 0)),
        ],
        out_specs=pl.BlockSpec((RB, H), lambda i: (i, 0)),
        out_shape=jax.ShapeDtypeStruct((N, H), jnp.float32),
    )(p, p, hws, dinv, b, Wd1, bd1, Wd2p, bd2p)


@jax.jit
def kernel(x, edge_index, edge_weight, W0, b0, W1, b1, W2, b2, Wd1, bd1,
           Wd2, bd2):
    r = edge_index[0].astype(jnp.int32)
    c = edge_index[1].astype(jnp.int32)
    w = edge_weight.astype(jnp.float32)
    pad = E_REAL - E
    rf = jnp.pad(jnp.pad(r, (0, pad)).reshape(NW, NCHUNK, CK),
                 ((0, 0), (0, NBUF), (0, 0))).reshape(-1)
    cf = jnp.pad(jnp.pad(c, (0, pad)).reshape(NW, NCHUNK, CK),
                 ((0, 0), (0, NBUF), (0, 0))).reshape(-1)
    wf = jnp.pad(jnp.pad(w, (0, pad)).reshape(NW, NCHUNK, CK),
                 ((0, 0), (0, NBUF), (0, 0))).reshape(-1)

    degp = _deg_kernel(rf, wf)
    d0 = degp[0, :N, None]
    d1 = degp[1, :N, None]

    hws, dinv = _mm1(x, W0, d0, d1)

    p = _agg_kernel(hws, rf, cf, wf)
    hws = _combine_mm(p, hws, dinv, b0.reshape(1, H), W1)

    p = _agg_kernel(hws, rf, cf, wf)
    hws = _combine_mm(p, hws, dinv, b1.reshape(1, H), W2)

    p = _agg_kernel(hws, rf, cf, wf)
    Wd2p = jnp.pad(Wd2, ((0, 0), (0, H - NUM_LABELS)))
    bd2p = jnp.pad(bd2, (0, H - NUM_LABELS)).reshape(1, H)
    out = _final(p, hws, dinv, b2.reshape(1, H), Wd1,
                 bd1.reshape(1, HID), Wd2p, bd2p)
    return out[:, :NUM_LABELS]


# speedup vs baseline: 2.1456x; 1.6466x over previous
import functools

import jax
import jax.numpy as jnp
from jax import lax
from jax.experimental import pallas as pl
from jax.experimental.pallas import tpu as pltpu
from jax.experimental.pallas import tpu_sc as plsc

N = 10000
D = 128
H = 128
HID = 256
NUM_LABELS = 7
E = 320000

NC = 2
NS = 16
NW = NC * NS
CK = 128
NCHUNK = 79
NCHUNK_T = NCHUNK
EPT = CK * NCHUNK
E_REAL = EPT * NW
NDEG = 10240
DEG_PT = NDEG // NS
NROW = 10240
ROWS_PT = NROW // NS

_mesh = plsc.VectorSubcoreMesh(core_axis_name="c", subcore_axis_name="s")


def _zero_vmem_2d(ref, nrows):
    z = jnp.zeros((16,), jnp.float32)

    def body(i, _):
        for j in range(8):
            ref[i, pl.ds(j * 16, 16)] = z
        return 0

    lax.fori_loop(0, nrows, body, 0)


def _zero_vmem_1d(ref, n):
    z = jnp.zeros((16,), jnp.float32)

    def body(i, _):
        ref[pl.ds(i * 16, 16)] = z
        return 0

    lax.fori_loop(0, n // 16, body, 0)


@functools.partial(
    pl.kernel,
    out_type=jax.ShapeDtypeStruct((NC, NDEG), jnp.float32),
    mesh=_mesh,
    scratch_types=[
        pltpu.VMEM_SHARED((NDEG,), jnp.float32),
        pltpu.VMEM((CK,), jnp.int32),
        pltpu.VMEM((CK,), jnp.float32),
        pltpu.VMEM((DEG_PT,), jnp.float32),
    ],
)
def _deg_kernel(rf_hbm, wf_hbm, out, deg_sp, ridx, wbuf, zbuf):
    cid = lax.axis_index("c")
    sid = lax.axis_index("s")
    wid = sid * NC + cid

    _zero_vmem_1d(zbuf, DEG_PT)
    pltpu.sync_copy(zbuf, deg_sp.at[pl.ds(sid * DEG_PT, DEG_PT)])
    plsc.subcore_barrier()

    def chunk(k, _):
        base = (wid * NCHUNK_T + k) * CK
        pltpu.sync_copy(rf_hbm.at[pl.ds(base, CK)], ridx)
        pltpu.sync_copy(wf_hbm.at[pl.ds(base, CK)], wbuf)
        pltpu.sync_copy(wbuf, deg_sp.at[ridx], add=True)
        return 0

    lax.fori_loop(0, NCHUNK, chunk, 0)
    plsc.subcore_barrier()
    pltpu.sync_copy(
        deg_sp.at[pl.ds(sid * DEG_PT, DEG_PT)],
        out.at[cid, pl.ds(sid * DEG_PT, DEG_PT)],
    )


@functools.partial(
    pl.kernel,
    out_type=jax.ShapeDtypeStruct((NC, NROW, H), jnp.float32),
    mesh=_mesh,
    scratch_types=[
        pltpu.VMEM_SHARED((NROW, H), jnp.float32),
        pltpu.VMEM((CK,), jnp.int32),
        pltpu.VMEM((CK,), jnp.int32),
        pltpu.VMEM((CK + 16,), jnp.float32),
        pltpu.VMEM((2 * CK, H), jnp.float32),
        pltpu.SemaphoreType.DMA,
    ],
)
def _agg_kernel(hws_hbm, rf_hbm, cf_hbm, wf_hbm, out, acc_sp, cidx0,
                ridx, wch, rows, sem0):
    cid = lax.axis_index("c")
    sid = lax.axis_index("s")
    wid = sid * NC + cid

    _zero_vmem_2d(rows, 2 * CK)
    base_row = sid * ROWS_PT
    pltpu.sync_copy(rows.at[pl.ds(0, 2 * CK), :],
                    acc_sp.at[pl.ds(base_row, 2 * CK), :])
    pltpu.sync_copy(rows.at[pl.ds(0, 2 * CK), :],
                    acc_sp.at[pl.ds(base_row + 2 * CK, 2 * CK), :])
    pltpu.sync_copy(rows.at[pl.ds(0, ROWS_PT - 4 * CK), :],
                    acc_sp.at[pl.ds(base_row + 4 * CK, ROWS_PT - 4 * CK), :])
    plsc.subcore_barrier()

    tbase = wid * NCHUNK_T * CK

    def rows_at(b):
        return rows.at[pl.ds(b * CK, CK), :]

    def chunk(k, _):
        base = tbase + k * CK
        pltpu.sync_copy(cf_hbm.at[pl.ds(base, CK)], cidx0)
        pltpu.async_copy(hws_hbm.at[cidx0], rows_at(0), sem0)
        pltpu.sync_copy(rf_hbm.at[pl.ds(base, CK)], ridx)
        pltpu.sync_copy(wf_hbm.at[pl.ds(base, CK)], wch.at[pl.ds(0, CK)])
        pltpu.make_async_copy(hws_hbm.at[cidx0], rows_at(0), sem0).wait()

        def scale(e, _):
            ws = wch[pl.ds(e, 16)][0]
            for j in range(8):
                sl = pl.ds(j * 16, 16)
                rows[e, sl] = rows[e, sl] * ws
            return 0

        lax.fori_loop(0, CK, scale, 0)
        pltpu.sync_copy(rows_at(0), acc_sp.at[ridx], add=True)
        return 0

    lax.fori_loop(0, NCHUNK, chunk, 0)
    plsc.subcore_barrier()

    for j in range(5):
        pltpu.sync_copy(acc_sp.at[pl.ds(base_row + j * CK, CK), :],
                        out.at[cid, pl.ds(base_row + j * CK, CK), :])


RB = 400
GRID = N // RB


def _mm1_body(x_ref, w_ref, d0_ref, d1_ref, hws_ref, dinv_ref):
    dinv = lax.rsqrt(d0_ref[...] + d1_ref[...] + 1.0)
    hw = jnp.dot(x_ref[...], w_ref[...], preferred_element_type=jnp.float32)
    hws_ref[...] = dinv * hw
    dinv_ref[...] = dinv


def _mm1(x, W0, d0, d1):
    return pl.pallas_call(
        _mm1_body,
        grid=(GRID,),
        in_specs=[
            pl.BlockSpec((RB, D), lambda i: (i, 0)),
            pl.BlockSpec((D, H), lambda i: (0, 0)),
            pl.BlockSpec((RB, 1), lambda i: (i, 0)),
            pl.BlockSpec((RB, 1), lambda i: (i, 0)),
        ],
        out_specs=[
            pl.BlockSpec((RB, H), lambda i: (i, 0)),
            pl.BlockSpec((RB, 1), lambda i: (i, 0)),
        ],
        out_shape=[
            jax.ShapeDtypeStruct((N, H), jnp.float32),
            jax.ShapeDtypeStruct((N, 1), jnp.float32),
        ],
    )(x, W0, d0, d1)


def _combine_mm_body(p0_ref, p1_ref, hws_ref, dinv_ref, b_ref, w_ref, out_ref):
    dinv = dinv_ref[...]
    h = jax.nn.relu(dinv * (p0_ref[0] + p1_ref[0] + hws_ref[...])
                    + b_ref[...])
    out_ref[...] = dinv * jnp.dot(h, w_ref[...],
                                  preferred_element_type=jnp.float32)


def _combine_mm(p, hws, dinv, b, W):
    return pl.pallas_call(
        _combine_mm_body,
        grid=(GRID,),
        in_specs=[
            pl.BlockSpec((1, RB, H), lambda i: (0, i, 0)),
            pl.BlockSpec((1, RB, H), lambda i: (1, i, 0)),
            pl.BlockSpec((RB, H), lambda i: (i, 0)),
            pl.BlockSpec((RB, 1), lambda i: (i, 0)),
            pl.BlockSpec((1, H), lambda i: (0, 0)),
            pl.BlockSpec((H, H), lambda i: (0, 0)),
        ],
        out_specs=pl.BlockSpec((RB, H), lambda i: (i, 0)),
        out_shape=jax.ShapeDtypeStruct((N, H), jnp.float32),
    )(p, p, hws, dinv, b, W)


def _final_body(p0_ref, p1_ref, hws_ref, dinv_ref, b_ref, wd1_ref, bd1_ref,
                wd2_ref, bd2_ref, out_ref):
    dinv = dinv_ref[...]
    h = jax.nn.relu(dinv * (p0_ref[0] + p1_ref[0] + hws_ref[...])
                    + b_ref[...])
    t = jax.nn.relu(jnp.dot(h, wd1_ref[...],
                            preferred_element_type=jnp.float32) + bd1_ref[...])
    out_ref[...] = jnp.dot(t, wd2_ref[...],
                           preferred_element_type=jnp.float32) + bd2_ref[...]


def _final(p, hws, dinv, b, Wd1, bd1, Wd2p, bd2p):
    return pl.pallas_call(
        _final_body,
        grid=(GRID,),
        in_specs=[
            pl.BlockSpec((1, RB, H), lambda i: (0, i, 0)),
            pl.BlockSpec((1, RB, H), lambda i: (1, i, 0)),
            pl.BlockSpec((RB, H), lambda i: (i, 0)),
            pl.BlockSpec((RB, 1), lambda i: (i, 0)),
            pl.BlockSpec((1, H), lambda i: (0, 0)),
            pl.BlockSpec((H, HID), lambda i: (0, 0)),
            pl.BlockSpec((1, HID), lambda i: (0, 0)),
            pl.BlockSpec((HID, H), lambda i: (0, 0)),
            pl.BlockSpec((1, H), lambda i: (0, 0)),
        ],
        out_specs=pl.BlockSpec((RB, H), lambda i: (i, 0)),
        out_shape=jax.ShapeDtypeStruct((N, H), jnp.float32),
    )(p, p, hws, dinv, b, Wd1, bd1, Wd2p, bd2p)


@jax.jit
def kernel(x, edge_index, edge_weight, W0, b0, W1, b1, W2, b2, Wd1, bd1,
           Wd2, bd2):
    r = edge_index[0].astype(jnp.int32)
    c = edge_index[1].astype(jnp.int32)
    w = edge_weight.astype(jnp.float32)
    pad = E_REAL - E
    rf = jnp.pad(r, (0, pad))
    cf = jnp.pad(c, (0, pad))
    wf = jnp.pad(w, (0, pad))

    degp = _deg_kernel(rf, wf)
    d0 = degp[0, :N, None]
    d1 = degp[1, :N, None]

    hws, dinv = _mm1(x, W0, d0, d1)

    p = _agg_kernel(hws, rf, cf, wf)
    hws = _combine_mm(p, hws, dinv, b0.reshape(1, H), W1)

    p = _agg_kernel(hws, rf, cf, wf)
    hws = _combine_mm(p, hws, dinv, b1.reshape(1, H), W2)

    p = _agg_kernel(hws, rf, cf, wf)
    Wd2p = jnp.pad(Wd2, ((0, 0), (0, H - NUM_LABELS)))
    bd2p = jnp.pad(bd2, (0, H - NUM_LABELS)).reshape(1, H)
    out = _final(p, hws, dinv, b2.reshape(1, H), Wd1,
                 bd1.reshape(1, HID), Wd2p, bd2p)
    return out[:, :NUM_LABELS]
